# speedup vs baseline: 2.8587x; 2.8587x over previous
import functools
import jax
import jax.numpy as jnp
from jax.experimental import pallas as pl


def _matmul_scale_kernel(x_ref, w_ref, s_ref, h_ref, hs_ref):
    h = jnp.dot(x_ref[...], w_ref[...], preferred_element_type=jnp.float32)
    h_ref[...] = h
    hs_ref[...] = h * s_ref[...]


def _matmul_scale(x, W, s):
    n, d_in = x.shape
    d_out = W.shape[1]
    blk = 1000
    return pl.pallas_call(
        _matmul_scale_kernel,
        grid=(n // blk,),
        in_specs=[
            pl.BlockSpec((blk, d_in), lambda i: (i, 0)),
            pl.BlockSpec((d_in, d_out), lambda i: (0, 0)),
            pl.BlockSpec((blk, 1), lambda i: (i, 0)),
        ],
        out_specs=[
            pl.BlockSpec((blk, d_out), lambda i: (i, 0)),
            pl.BlockSpec((blk, d_out), lambda i: (i, 0)),
        ],
        out_shape=[
            jax.ShapeDtypeStruct((n, d_out), jnp.float32),
            jax.ShapeDtypeStruct((n, d_out), jnp.float32),
        ],
    )(x, W, s)


def _combine_kernel(seg_ref, h_ref, s_ref, b_ref, o_ref, *, relu):
    s = s_ref[...]
    o = s * seg_ref[...] + (s * s) * h_ref[...] + b_ref[...]
    if relu:
        o = jnp.maximum(o, 0.0)
    o_ref[...] = o


def _combine(seg, h, s, b, relu):
    n, d = seg.shape
    blk = 1000
    return pl.pallas_call(
        functools.partial(_combine_kernel, relu=relu),
        grid=(n // blk,),
        in_specs=[
            pl.BlockSpec((blk, d), lambda i: (i, 0)),
            pl.BlockSpec((blk, d), lambda i: (i, 0)),
            pl.BlockSpec((blk, 1), lambda i: (i, 0)),
            pl.BlockSpec((1, d), lambda i: (0, 0)),
        ],
        out_specs=pl.BlockSpec((blk, d), lambda i: (i, 0)),
        out_shape=jax.ShapeDtypeStruct((n, d), jnp.float32),
    )(seg, h, s, b.reshape(1, d))


def kernel(x, edge_index, W1, b1, W2, b2):
    n = x.shape[0]
    src = edge_index[0]
    dst = edge_index[1]
    ones = jnp.ones(src.shape[0], dtype=jnp.float32)
    deg = jax.ops.segment_sum(ones, dst, num_segments=n) + 1.0
    dinv = jax.lax.rsqrt(deg).reshape(n, 1)

    h1, h1s = _matmul_scale(x, W1, dinv)
    seg1 = jax.ops.segment_sum(h1s[src], dst, num_segments=n)
    z1 = _combine(seg1, h1, dinv, b1, relu=True)

    h2, h2s = _matmul_scale(z1, W2, dinv)
    seg2 = jax.ops.segment_sum(h2s[src], dst, num_segments=n)
    out = _combine(seg2, h2, dinv, b2, relu=False)
    return out


# speedup vs baseline: 13.9043x; 4.8638x over previous
import jax
import jax.numpy as jnp
from jax import lax
from jax.experimental import pallas as pl
from jax.experimental.pallas import tpu as pltpu
from jax.experimental.pallas import tpu_sc as plsc

_N = 10000
_E = 320000
_NC = 2
_NS = 16
_B = 80
_DC = 64
_NPAD = 10240
_ROWS_PER_TILE = _NPAD // _NS
_WB = 128
_NBLK = _E // (_NS * _B)

_mesh = plsc.VectorSubcoreMesh(core_axis_name="c", subcore_axis_name="s")
_cparams = pltpu.CompilerParams(use_tc_tiling_on_sc=False)


def _zero_fill(buf):
    rows, cols = buf.shape
    zeros = jnp.zeros((16,), jnp.float32)

    @pl.loop(0, rows)
    def _(r):
        for c in range(cols // 16):
            buf[r, pl.ds(c * 16, 16)] = zeros


def _zero_acc_slice(stage, acc, base):
    for k in range(_ROWS_PER_TILE // _WB):
        pltpu.sync_copy(stage, acc.at[pl.ds(base + k * _WB, _WB)])


def _writeback(stage, acc, base, out_ref):
    for k in range(_ROWS_PER_TILE // _WB):
        sl = pl.ds(base + k * _WB, _WB)
        pltpu.sync_copy(acc.at[sl], stage)
        pltpu.sync_copy(stage, out_ref.at[sl])


def _deg_body(dst_hbm, out_hbm, didx, ones_v, stage, acc):
    cid = lax.axis_index("c")
    sid = lax.axis_index("s")
    wid = sid * _NC + cid
    nblk = _E // (_NC * _NS * _B)

    pltpu.sync_copy(dst_hbm.at[wid], didx)
    ones = jnp.ones((16,), jnp.float32)

    @pl.loop(0, _B)
    def _(r):
        ones_v[r, pl.ds(0, 16)] = ones

    _zero_fill(stage)
    base = sid * _ROWS_PER_TILE
    _zero_acc_slice(stage, acc, base)
    plsc.subcore_barrier()

    @pl.loop(0, nblk)
    def _(j):
        pltpu.sync_copy(ones_v, acc.at[didx.at[j]], add=True)

    plsc.subcore_barrier()

    _writeback(stage, acc, base, out_hbm.at[cid])


def _deg_pass(dst):
    dst3 = dst.reshape(_NC * _NS, _E // (_NC * _NS * _B), _B)
    fn = pl.kernel(
        _deg_body,
        out_type=jax.ShapeDtypeStruct((_NC, _NPAD, 16), jnp.float32),
        mesh=_mesh,
        compiler_params=_cparams,
        scratch_types=[
            pltpu.VMEM((_E // (_NC * _NS * _B), _B), jnp.int32),
            pltpu.VMEM((_B, 16), jnp.float32),
            pltpu.VMEM((_WB, 16), jnp.float32),
            pltpu.VMEM_SHARED((_NPAD, 16), jnp.float32),
        ],
    )
    return fn(dst3)


def _scatter_body(tab_hbm, src_hbm, dst_hbm, out_hbm,
                  sidx, didx, rows0, rows1, stage, acc, sem0, sem1):
    cid = lax.axis_index("c")
    sid = lax.axis_index("s")
    base = sid * _ROWS_PER_TILE

    pltpu.sync_copy(src_hbm.at[cid].at[sid], sidx)
    pltpu.sync_copy(dst_hbm.at[sid], didx)

    _zero_fill(stage)
    _zero_acc_slice(stage, acc, base)
    plsc.subcore_barrier()

    pltpu.make_async_copy(tab_hbm.at[sidx.at[0]], rows0, sem0).start()

    @pl.loop(0, _NBLK, step=2)
    def _(j):
        pltpu.make_async_copy(tab_hbm.at[sidx.at[j]], rows0, sem0).wait()
        pltpu.make_async_copy(tab_hbm.at[sidx.at[j + 1]], rows1, sem1).start()
        pltpu.sync_copy(rows0, acc.at[didx.at[j]], add=True)

        pltpu.make_async_copy(tab_hbm.at[sidx.at[j + 1]], rows1, sem1).wait()

        @pl.when(j + 2 < _NBLK)
        def _():
            pltpu.make_async_copy(tab_hbm.at[sidx.at[j + 2]], rows0,
                                  sem0).start()

        pltpu.sync_copy(rows1, acc.at[didx.at[j + 1]], add=True)

    plsc.subcore_barrier()

    _writeback(stage, acc, base, out_hbm.at[cid])


def _scatter_pass(tables, src4, dst3):
    fn = pl.kernel(
        _scatter_body,
        out_type=jax.ShapeDtypeStruct((_NC, _NPAD, _DC), jnp.float32),
        mesh=_mesh,
        compiler_params=_cparams,
        scratch_types=[
            pltpu.VMEM((_NBLK, _B), jnp.int32),
            pltpu.VMEM((_NBLK, _B), jnp.int32),
            pltpu.VMEM((_B, _DC), jnp.float32),
            pltpu.VMEM((_B, _DC), jnp.float32),
            pltpu.VMEM((_WB, _DC), jnp.float32),
            pltpu.VMEM_SHARED((_NPAD, _DC), jnp.float32),
            pltpu.SemaphoreType.DMA,
            pltpu.SemaphoreType.DMA,
        ],
    )
    return fn(tables.reshape(_NC * _N, _DC), src4, dst3)


_BLK = 1000


def _mm_kernel(x_ref, w_ref, o_ref):
    o_ref[...] = jnp.dot(x_ref[...], w_ref[...],
                         preferred_element_type=jnp.float32)


def _matmul(x, W):
    n, d_in = x.shape
    d_out = W.shape[1]
    return pl.pallas_call(
        _mm_kernel,
        grid=(n // _BLK,),
        in_specs=[
            pl.BlockSpec((_BLK, d_in), lambda i: (i, 0)),
            pl.BlockSpec((d_in, d_out), lambda i: (0, 0)),
        ],
        out_specs=pl.BlockSpec((_BLK, d_out), lambda i: (i, 0)),
        out_shape=jax.ShapeDtypeStruct((n, d_out), jnp.float32),
    )(x, W)


def _scale1_kernel(degp_ref, h_ref, s_ref, hsa_ref, hsb_ref):
    deg = degp_ref[0, :, :1] + degp_ref[1, :, :1] + 1.0
    s = jax.lax.rsqrt(deg)
    s_ref[...] = s
    for q in range(2):
        hsa_ref[q] = h_ref[:, q * 64:(q + 1) * 64] * s
        hsb_ref[q] = h_ref[:, 128 + q * 64:128 + (q + 1) * 64] * s


def _scale1(degp, h1):
    n = h1.shape[0]
    chunk_pair = jax.ShapeDtypeStruct((2, n, 64), jnp.float32)
    return pl.pallas_call(
        _scale1_kernel,
        grid=(n // _BLK,),
        in_specs=[
            pl.BlockSpec((2, _BLK, 16), lambda i: (0, i, 0)),
            pl.BlockSpec((_BLK, 256), lambda i: (i, 0)),
        ],
        out_specs=[
            pl.BlockSpec((_BLK, 1), lambda i: (i, 0)),
            pl.BlockSpec((2, _BLK, 64), lambda i: (0, i, 0)),
            pl.BlockSpec((2, _BLK, 64), lambda i: (0, i, 0)),
        ],
        out_shape=[
            jax.ShapeDtypeStruct((n, 1), jnp.float32),
            chunk_pair,
            chunk_pair,
        ],
    )(degp, h1)


def _mid_kernel(sega_ref, segb_ref, h1_ref, s_ref, b1_ref, w2_ref,
                h2_ref, hs_ref):
    s = s_ref[...]
    s2 = s * s
    zs = []
    for q, seg in enumerate([sega_ref[0], sega_ref[1],
                             segb_ref[0], segb_ref[1]]):
        z = (s * seg + s2 * h1_ref[:, q * 64:(q + 1) * 64]
             + b1_ref[:, q * 64:(q + 1) * 64])
        zs.append(jnp.maximum(z, 0.0))
    z = jnp.concatenate(zs, axis=1)
    h2 = jnp.dot(z, w2_ref[...], preferred_element_type=jnp.float32)
    h2_ref[...] = h2
    hs_ref[0] = h2[:, :64] * s
    hs_ref[1] = h2[:, 64:] * s


def _mid(seg1a, seg1b, h1, s, b1, W2):
    n = h1.shape[0]
    return pl.pallas_call(
        _mid_kernel,
        grid=(n // _BLK,),
        in_specs=[
            pl.BlockSpec((2, _BLK, 64), lambda i: (0, i, 0)),
            pl.BlockSpec((2, _BLK, 64), lambda i: (0, i, 0)),
            pl.BlockSpec((_BLK, 256), lambda i: (i, 0)),
            pl.BlockSpec((_BLK, 1), lambda i: (i, 0)),
            pl.BlockSpec((1, 256), lambda i: (0, 0)),
            pl.BlockSpec((256, 128), lambda i: (0, 0)),
        ],
        out_specs=[
            pl.BlockSpec((_BLK, 128), lambda i: (i, 0)),
            pl.BlockSpec((2, _BLK, 64), lambda i: (0, i, 0)),
        ],
        out_shape=[
            jax.ShapeDtypeStruct((n, 128), jnp.float32),
            jax.ShapeDtypeStruct((2, n, 64), jnp.float32),
        ],
    )(seg1a, seg1b, h1, s, b1.reshape(1, 256), W2)


def _final_kernel(seg_ref, h2_ref, s_ref, b2_ref, o_ref):
    s = s_ref[...]
    s2 = s * s
    o_lo = s * seg_ref[0] + s2 * h2_ref[:, :64] + b2_ref[:, :64]
    o_hi = s * seg_ref[1] + s2 * h2_ref[:, 64:] + b2_ref[:, 64:]
    o_ref[...] = jnp.concatenate([o_lo, o_hi], axis=1)


def _final(seg2, h2, s, b2):
    n = h2.shape[0]
    return pl.pallas_call(
        _final_kernel,
        grid=(n // _BLK,),
        in_specs=[
            pl.BlockSpec((2, _BLK, 64), lambda i: (0, i, 0)),
            pl.BlockSpec((_BLK, 128), lambda i: (i, 0)),
            pl.BlockSpec((_BLK, 1), lambda i: (i, 0)),
            pl.BlockSpec((1, 128), lambda i: (0, 0)),
        ],
        out_specs=pl.BlockSpec((_BLK, 128), lambda i: (i, 0)),
        out_shape=jax.ShapeDtypeStruct((n, 128), jnp.float32),
    )(seg2, h2, s, b2.reshape(1, 128))


def kernel(x, edge_index, W1, b1, W2, b2):
    src = edge_index[0]
    dst = edge_index[1]

    offs = (jnp.arange(_NC, dtype=jnp.int32) * _N)[:, None, None, None]
    src4 = src.reshape(1, _NS, _NBLK, _B) + offs
    dst3 = dst.reshape(_NS, _NBLK, _B)

    degp = _deg_pass(dst)
    h1 = _matmul(x, W1)
    s, h1sa, h1sb = _scale1(degp, h1)
    seg1a = _scatter_pass(h1sa, src4, dst3)
    seg1b = _scatter_pass(h1sb, src4, dst3)
    h2, h2s = _mid(seg1a, seg1b, h1, s, b1, W2)
    seg2 = _scatter_pass(h2s, src4, dst3)
    return _final(seg2, h2, s, b2)


# speedup vs baseline: 22.9308x; 1.6492x over previous
import jax
import jax.numpy as jnp
from jax import lax
from jax.experimental import pallas as pl
from jax.experimental.pallas import tpu as pltpu
from jax.experimental.pallas import tpu_sc as plsc

_N = 10000
_E = 320000
_NC = 2
_NS = 16
_B = 80
_DC = 64
_NPAD = 10240
_ROWS_PER_TILE = _NPAD // _NS
_WB = 128
_NBLK = _E // (_NS * _B)

_mesh = plsc.VectorSubcoreMesh(core_axis_name="c", subcore_axis_name="s")
_cparams = pltpu.CompilerParams(use_tc_tiling_on_sc=False)


def _zero_fill(buf):
    rows, cols = buf.shape
    zeros = jnp.zeros((16,), jnp.float32)

    @pl.loop(0, rows)
    def _(r):
        for c in range(cols // 16):
            buf[r, pl.ds(c * 16, 16)] = zeros


def _zero_acc_slice(stage, acc, base):
    for k in range(_ROWS_PER_TILE // _WB):
        pltpu.sync_copy(stage, acc.at[pl.ds(base + k * _WB, _WB)])


def _writeback(stage, acc, base, out_ref):
    for k in range(_ROWS_PER_TILE // _WB):
        sl = pl.ds(base + k * _WB, _WB)
        pltpu.sync_copy(acc.at[sl], stage)
        pltpu.sync_copy(stage, out_ref.at[sl])


def _deg_body(dst_hbm, out_hbm, didx, ones_v, stage, acc):
    cid = lax.axis_index("c")
    sid = lax.axis_index("s")
    wid = sid * _NC + cid
    nblk = _E // (_NC * _NS * _B)

    pltpu.sync_copy(dst_hbm.at[wid], didx)
    ones = jnp.ones((16,), jnp.float32)

    @pl.loop(0, _B)
    def _(r):
        ones_v[r, pl.ds(0, 16)] = ones

    _zero_fill(stage)
    base = sid * _ROWS_PER_TILE
    _zero_acc_slice(stage, acc, base)
    plsc.subcore_barrier()

    @pl.loop(0, nblk)
    def _(j):
        pltpu.sync_copy(ones_v, acc.at[didx.at[j]], add=True)

    plsc.subcore_barrier()

    _writeback(stage, acc, base, out_hbm.at[cid])


def _deg_pass(dst):
    dst3 = dst.reshape(_NC * _NS, _E // (_NC * _NS * _B), _B)
    fn = pl.kernel(
        _deg_body,
        out_type=jax.ShapeDtypeStruct((_NC, _NPAD, 16), jnp.float32),
        mesh=_mesh,
        compiler_params=_cparams,
        scratch_types=[
            pltpu.VMEM((_E // (_NC * _NS * _B), _B), jnp.int32),
            pltpu.VMEM((_B, 16), jnp.float32),
            pltpu.VMEM((_WB, 16), jnp.float32),
            pltpu.VMEM_SHARED((_NPAD, 16), jnp.float32),
        ],
    )
    return fn(dst3)


def _gat(tab_hbm, sidx, j, buf, sem):
    return pltpu.make_async_copy(tab_hbm.at[sidx.at[j]], buf, sem)


def _sca(acc, didx, j, buf, sem):
    return pltpu.make_async_copy(buf, acc.at[didx.at[j]], sem)


def _scatter_body(tab_hbm, src_hbm, dst_hbm, out_hbm,
                  sidx, didx, r0, r1, r2, r3, stage, acc,
                  g0, g1, g2, g3, s0, s1, s2, s3):
    cid = lax.axis_index("c")
    sid = lax.axis_index("s")
    base = sid * _ROWS_PER_TILE
    rows = [r0, r1, r2, r3]
    gsem = [g0, g1, g2, g3]
    ssem = [s0, s1, s2, s3]

    pltpu.sync_copy(src_hbm.at[cid].at[sid], sidx)
    pltpu.sync_copy(dst_hbm.at[sid], didx)

    _zero_fill(stage)
    _zero_acc_slice(stage, acc, base)
    plsc.subcore_barrier()

    for b in range(4):
        _gat(tab_hbm, sidx, b, rows[b], gsem[b]).start()

    _TAIL = _NBLK % 4
    _MAIN = _NBLK - _TAIL

    @pl.loop(0, _MAIN, step=4)
    def _(j):
        for b in range(4):
            _gat(tab_hbm, sidx, j + b, rows[b], gsem[b]).wait()
            _sca(acc, didx, j + b, rows[b], ssem[b]).start(add=True)
        for b in range(4):
            _sca(acc, didx, j + b, rows[b], ssem[b]).wait()

            @pl.when(j + b + 4 < _NBLK)
            def _():
                _gat(tab_hbm, sidx, j + b + 4, rows[b], gsem[b]).start()

    for b in range(_TAIL):
        _gat(tab_hbm, sidx, _MAIN + b, rows[b], gsem[b]).wait()
        _sca(acc, didx, _MAIN + b, rows[b], ssem[b]).start(add=True)
    for b in range(_TAIL):
        _sca(acc, didx, _MAIN + b, rows[b], ssem[b]).wait()

    plsc.subcore_barrier()

    _writeback(stage, acc, base, out_hbm.at[cid])


def _scatter_pass(tables, src4, dst3):
    fn = pl.kernel(
        _scatter_body,
        out_type=jax.ShapeDtypeStruct((_NC, _NPAD, _DC), jnp.float32),
        mesh=_mesh,
        compiler_params=_cparams,
        scratch_types=[
            pltpu.VMEM((_NBLK, _B), jnp.int32),
            pltpu.VMEM((_NBLK, _B), jnp.int32),
            pltpu.VMEM((_B, _DC), jnp.float32),
            pltpu.VMEM((_B, _DC), jnp.float32),
            pltpu.VMEM((_B, _DC), jnp.float32),
            pltpu.VMEM((_B, _DC), jnp.float32),
            pltpu.VMEM((_WB, _DC), jnp.float32),
            pltpu.VMEM_SHARED((_NPAD, _DC), jnp.float32),
        ] + [pltpu.SemaphoreType.DMA] * 8,
    )
    return fn(tables.reshape(_NC * _N, _DC), src4, dst3)


_BLK = 1000


def _mm_kernel(x_ref, w_ref, o_ref):
    o_ref[...] = jnp.dot(x_ref[...], w_ref[...],
                         preferred_element_type=jnp.float32)


def _matmul(x, W):
    n, d_in = x.shape
    d_out = W.shape[1]
    return pl.pallas_call(
        _mm_kernel,
        grid=(n // _BLK,),
        in_specs=[
            pl.BlockSpec((_BLK, d_in), lambda i: (i, 0)),
            pl.BlockSpec((d_in, d_out), lambda i: (0, 0)),
        ],
        out_specs=pl.BlockSpec((_BLK, d_out), lambda i: (i, 0)),
        out_shape=jax.ShapeDtypeStruct((n, d_out), jnp.float32),
    )(x, W)


def _scale1_kernel(degp_ref, h_ref, s_ref, hsa_ref, hsb_ref):
    deg = degp_ref[0, :, :1] + degp_ref[1, :, :1] + 1.0
    s = jax.lax.rsqrt(deg)
    s_ref[...] = s
    for q in range(2):
        hsa_ref[q] = h_ref[:, q * 64:(q + 1) * 64] * s
        hsb_ref[q] = h_ref[:, 128 + q * 64:128 + (q + 1) * 64] * s


def _scale1(degp, h1):
    n = h1.shape[0]
    chunk_pair = jax.ShapeDtypeStruct((2, n, 64), jnp.float32)
    return pl.pallas_call(
        _scale1_kernel,
        grid=(n // _BLK,),
        in_specs=[
            pl.BlockSpec((2, _BLK, 16), lambda i: (0, i, 0)),
            pl.BlockSpec((_BLK, 256), lambda i: (i, 0)),
        ],
        out_specs=[
            pl.BlockSpec((_BLK, 1), lambda i: (i, 0)),
            pl.BlockSpec((2, _BLK, 64), lambda i: (0, i, 0)),
            pl.BlockSpec((2, _BLK, 64), lambda i: (0, i, 0)),
        ],
        out_shape=[
            jax.ShapeDtypeStruct((n, 1), jnp.float32),
            chunk_pair,
            chunk_pair,
        ],
    )(degp, h1)


def _mid_kernel(sega_ref, segb_ref, h1_ref, s_ref, b1_ref, w2_ref,
                h2_ref, hs_ref):
    s = s_ref[...]
    s2 = s * s
    zs = []
    for q, seg in enumerate([sega_ref[0], sega_ref[1],
                             segb_ref[0], segb_ref[1]]):
        z = (s * seg + s2 * h1_ref[:, q * 64:(q + 1) * 64]
             + b1_ref[:, q * 64:(q + 1) * 64])
        zs.append(jnp.maximum(z, 0.0))
    z = jnp.concatenate(zs, axis=1)
    h2 = jnp.dot(z, w2_ref[...], preferred_element_type=jnp.float32)
    h2_ref[...] = h2
    hs_ref[0] = h2[:, :64] * s
    hs_ref[1] = h2[:, 64:] * s


def _mid(seg1a, seg1b, h1, s, b1, W2):
    n = h1.shape[0]
    return pl.pallas_call(
        _mid_kernel,
        grid=(n // _BLK,),
        in_specs=[
            pl.BlockSpec((2, _BLK, 64), lambda i: (0, i, 0)),
            pl.BlockSpec((2, _BLK, 64), lambda i: (0, i, 0)),
            pl.BlockSpec((_BLK, 256), lambda i: (i, 0)),
            pl.BlockSpec((_BLK, 1), lambda i: (i, 0)),
            pl.BlockSpec((1, 256), lambda i: (0, 0)),
            pl.BlockSpec((256, 128), lambda i: (0, 0)),
        ],
        out_specs=[
            pl.BlockSpec((_BLK, 128), lambda i: (i, 0)),
            pl.BlockSpec((2, _BLK, 64), lambda i: (0, i, 0)),
        ],
        out_shape=[
            jax.ShapeDtypeStruct((n, 128), jnp.float32),
            jax.ShapeDtypeStruct((2, n, 64), jnp.float32),
        ],
    )(seg1a, seg1b, h1, s, b1.reshape(1, 256), W2)


def _final_kernel(seg_ref, h2_ref, s_ref, b2_ref, o_ref):
    s = s_ref[...]
    s2 = s * s
    o_lo = s * seg_ref[0] + s2 * h2_ref[:, :64] + b2_ref[:, :64]
    o_hi = s * seg_ref[1] + s2 * h2_ref[:, 64:] + b2_ref[:, 64:]
    o_ref[...] = jnp.concatenate([o_lo, o_hi], axis=1)


def _final(seg2, h2, s, b2):
    n = h2.shape[0]
    return pl.pallas_call(
        _final_kernel,
        grid=(n // _BLK,),
        in_specs=[
            pl.BlockSpec((2, _BLK, 64), lambda i: (0, i, 0)),
            pl.BlockSpec((_BLK, 128), lambda i: (i, 0)),
            pl.BlockSpec((_BLK, 1), lambda i: (i, 0)),
            pl.BlockSpec((1, 128), lambda i: (0, 0)),
        ],
        out_specs=pl.BlockSpec((_BLK, 128), lambda i: (i, 0)),
        out_shape=jax.ShapeDtypeStruct((n, 128), jnp.float32),
    )(seg2, h2, s, b2.reshape(1, 128))


def kernel(x, edge_index, W1, b1, W2, b2):
    src = edge_index[0]
    dst = edge_index[1]

    offs = (jnp.arange(_NC, dtype=jnp.int32) * _N)[:, None, None, None]
    src4 = src.reshape(1, _NS, _NBLK, _B) + offs
    dst3 = dst.reshape(_NS, _NBLK, _B)

    degp = _deg_pass(dst)
    h1 = _matmul(x, W1)
    s, h1sa, h1sb = _scale1(degp, h1)
    seg1a = _scatter_pass(h1sa, src4, dst3)
    seg1b = _scatter_pass(h1sb, src4, dst3)
    h2, h2s = _mid(seg1a, seg1b, h1, s, b1, W2)
    seg2 = _scatter_pass(h2s, src4, dst3)
    return _final(seg2, h2, s, b2)


# speedup vs baseline: 24.4517x; 1.0663x over previous
import jax
import jax.numpy as jnp
from jax import lax
from jax.experimental import pallas as pl
from jax.experimental.pallas import tpu as pltpu
from jax.experimental.pallas import tpu_sc as plsc

_N = 10000
_E = 320000
_NC = 2
_NS = 16
_B = 80
_DC = 64
_NPAD = 10240
_ROWS_PER_TILE = _NPAD // _NS
_WB = 128
_NBLK = _E // (_NS * _B)
_RING = 8

_mesh = plsc.VectorSubcoreMesh(core_axis_name="c", subcore_axis_name="s")
_cparams = pltpu.CompilerParams(use_tc_tiling_on_sc=False)


def _zero_fill(buf):
    rows, cols = buf.shape
    zeros = jnp.zeros((16,), jnp.float32)

    @pl.loop(0, rows)
    def _(r):
        for c in range(cols // 16):
            buf[r, pl.ds(c * 16, 16)] = zeros


def _zero_acc_slice(stage, acc, base):
    for k in range(_ROWS_PER_TILE // _WB):
        pltpu.sync_copy(stage, acc.at[pl.ds(base + k * _WB, _WB)])


def _writeback(stage, acc, base, out_ref):
    for k in range(_ROWS_PER_TILE // _WB):
        sl = pl.ds(base + k * _WB, _WB)
        pltpu.sync_copy(acc.at[sl], stage)
        pltpu.sync_copy(stage, out_ref.at[sl])


def _deg_body(dst_hbm, out_hbm, didx, ones_v, stage, acc):
    cid = lax.axis_index("c")
    sid = lax.axis_index("s")
    wid = sid * _NC + cid
    nblk = _E // (_NC * _NS * _B)

    pltpu.sync_copy(dst_hbm.at[wid], didx)
    ones = jnp.ones((16,), jnp.float32)

    @pl.loop(0, _B)
    def _(r):
        ones_v[r, pl.ds(0, 16)] = ones

    _zero_fill(stage)
    base = sid * _ROWS_PER_TILE
    _zero_acc_slice(stage, acc, base)
    plsc.subcore_barrier()

    @pl.loop(0, nblk)
    def _(j):
        pltpu.sync_copy(ones_v, acc.at[didx.at[j]], add=True)

    plsc.subcore_barrier()

    _writeback(stage, acc, base, out_hbm.at[cid])


def _deg_pass(dst):
    dst3 = dst.reshape(_NC * _NS, _E // (_NC * _NS * _B), _B)
    fn = pl.kernel(
        _deg_body,
        out_type=jax.ShapeDtypeStruct((_NC, _NPAD, 16), jnp.float32),
        mesh=_mesh,
        compiler_params=_cparams,
        scratch_types=[
            pltpu.VMEM((_E // (_NC * _NS * _B), _B), jnp.int32),
            pltpu.VMEM((_B, 16), jnp.float32),
            pltpu.VMEM((_WB, 16), jnp.float32),
            pltpu.VMEM_SHARED((_NPAD, 16), jnp.float32),
        ],
    )
    return fn(dst3)


def _gat(tab_hbm, sidx, j, buf, sem):
    return pltpu.make_async_copy(tab_hbm.at[sidx.at[j]], buf, sem)


def _sca(acc, didx, j, buf, sem):
    return pltpu.make_async_copy(buf, acc.at[didx.at[j]], sem)


def _scatter_body(tab_hbm, src_hbm, dst_hbm, out_hbm, *refs):
    sidx, didx = refs[0], refs[1]
    rows = list(refs[2:2 + _RING])
    stage = refs[2 + _RING]
    acc = refs[3 + _RING]
    gsem = list(refs[4 + _RING:4 + 2 * _RING])
    ssem = list(refs[4 + 2 * _RING:4 + 3 * _RING])
    cid = lax.axis_index("c")
    sid = lax.axis_index("s")
    base = sid * _ROWS_PER_TILE

    pltpu.sync_copy(src_hbm.at[cid].at[sid], sidx)
    pltpu.sync_copy(dst_hbm.at[sid], didx)

    _zero_fill(stage)
    _zero_acc_slice(stage, acc, base)
    plsc.subcore_barrier()

    for b in range(_RING):
        _gat(tab_hbm, sidx, b, rows[b], gsem[b]).start()

    _TAIL = _NBLK % _RING
    _MAIN = _NBLK - _TAIL

    @pl.loop(0, _MAIN, step=_RING)
    def _(j):
        for b in range(_RING):
            _gat(tab_hbm, sidx, j + b, rows[b], gsem[b]).wait()
            _sca(acc, didx, j + b, rows[b], ssem[b]).start(add=True)
        for b in range(_RING):
            _sca(acc, didx, j + b, rows[b], ssem[b]).wait()

            @pl.when(j + b + _RING < _NBLK)
            def _():
                _gat(tab_hbm, sidx, j + b + _RING, rows[b], gsem[b]).start()

    for b in range(_TAIL):
        _gat(tab_hbm, sidx, _MAIN + b, rows[b], gsem[b]).wait()
        _sca(acc, didx, _MAIN + b, rows[b], ssem[b]).start(add=True)
    for b in range(_TAIL):
        _sca(acc, didx, _MAIN + b, rows[b], ssem[b]).wait()

    plsc.subcore_barrier()

    _writeback(stage, acc, base, out_hbm.at[cid])


def _scatter_pass(tables, src4, dst3):
    fn = pl.kernel(
        _scatter_body,
        out_type=jax.ShapeDtypeStruct((_NC, _NPAD, _DC), jnp.float32),
        mesh=_mesh,
        compiler_params=_cparams,
        scratch_types=[
            pltpu.VMEM((_NBLK, _B), jnp.int32),
            pltpu.VMEM((_NBLK, _B), jnp.int32),
        ] + [pltpu.VMEM((_B, _DC), jnp.float32)] * _RING + [
            pltpu.VMEM((_WB, _DC), jnp.float32),
            pltpu.VMEM_SHARED((_NPAD, _DC), jnp.float32),
        ] + [pltpu.SemaphoreType.DMA] * (2 * _RING),
    )
    return fn(tables.reshape(_NC * _N, _DC), src4, dst3)


_BLK = 1000


def _mm_kernel(x_ref, w_ref, o_ref):
    o_ref[...] = jnp.dot(x_ref[...], w_ref[...],
                         preferred_element_type=jnp.float32)


def _matmul(x, W):
    n, d_in = x.shape
    d_out = W.shape[1]
    return pl.pallas_call(
        _mm_kernel,
        grid=(n // _BLK,),
        in_specs=[
            pl.BlockSpec((_BLK, d_in), lambda i: (i, 0)),
            pl.BlockSpec((d_in, d_out), lambda i: (0, 0)),
        ],
        out_specs=pl.BlockSpec((_BLK, d_out), lambda i: (i, 0)),
        out_shape=jax.ShapeDtypeStruct((n, d_out), jnp.float32),
    )(x, W)


def _scale1_kernel(degp_ref, h_ref, s_ref, hsa_ref, hsb_ref):
    deg = degp_ref[0, :, :1] + degp_ref[1, :, :1] + 1.0
    s = jax.lax.rsqrt(deg)
    s_ref[...] = s
    for q in range(2):
        hsa_ref[q] = h_ref[:, q * 64:(q + 1) * 64] * s
        hsb_ref[q] = h_ref[:, 128 + q * 64:128 + (q + 1) * 64] * s


def _scale1(degp, h1):
    n = h1.shape[0]
    chunk_pair = jax.ShapeDtypeStruct((2, n, 64), jnp.float32)
    return pl.pallas_call(
        _scale1_kernel,
        grid=(n // _BLK,),
        in_specs=[
            pl.BlockSpec((2, _BLK, 16), lambda i: (0, i, 0)),
            pl.BlockSpec((_BLK, 256), lambda i: (i, 0)),
        ],
        out_specs=[
            pl.BlockSpec((_BLK, 1), lambda i: (i, 0)),
            pl.BlockSpec((2, _BLK, 64), lambda i: (0, i, 0)),
            pl.BlockSpec((2, _BLK, 64), lambda i: (0, i, 0)),
        ],
        out_shape=[
            jax.ShapeDtypeStruct((n, 1), jnp.float32),
            chunk_pair,
            chunk_pair,
        ],
    )(degp, h1)


def _mid_kernel(sega_ref, segb_ref, h1_ref, s_ref, b1_ref, w2_ref,
                h2_ref, hs_ref):
    s = s_ref[...]
    s2 = s * s
    zs = []
    for q, seg in enumerate([sega_ref[0], sega_ref[1],
                             segb_ref[0], segb_ref[1]]):
        z = (s * seg + s2 * h1_ref[:, q * 64:(q + 1) * 64]
             + b1_ref[:, q * 64:(q + 1) * 64])
        zs.append(jnp.maximum(z, 0.0))
    z = jnp.concatenate(zs, axis=1)
    h2 = jnp.dot(z, w2_ref[...], preferred_element_type=jnp.float32)
    h2_ref[...] = h2
    hs_ref[0] = h2[:, :64] * s
    hs_ref[1] = h2[:, 64:] * s


def _mid(seg1a, seg1b, h1, s, b1, W2):
    n = h1.shape[0]
    return pl.pallas_call(
        _mid_kernel,
        grid=(n // _BLK,),
        in_specs=[
            pl.BlockSpec((2, _BLK, 64), lambda i: (0, i, 0)),
            pl.BlockSpec((2, _BLK, 64), lambda i: (0, i, 0)),
            pl.BlockSpec((_BLK, 256), lambda i: (i, 0)),
            pl.BlockSpec((_BLK, 1), lambda i: (i, 0)),
            pl.BlockSpec((1, 256), lambda i: (0, 0)),
            pl.BlockSpec((256, 128), lambda i: (0, 0)),
        ],
        out_specs=[
            pl.BlockSpec((_BLK, 128), lambda i: (i, 0)),
            pl.BlockSpec((2, _BLK, 64), lambda i: (0, i, 0)),
        ],
        out_shape=[
            jax.ShapeDtypeStruct((n, 128), jnp.float32),
            jax.ShapeDtypeStruct((2, n, 64), jnp.float32),
        ],
    )(seg1a, seg1b, h1, s, b1.reshape(1, 256), W2)


def _final_kernel(seg_ref, h2_ref, s_ref, b2_ref, o_ref):
    s = s_ref[...]
    s2 = s * s
    o_lo = s * seg_ref[0] + s2 * h2_ref[:, :64] + b2_ref[:, :64]
    o_hi = s * seg_ref[1] + s2 * h2_ref[:, 64:] + b2_ref[:, 64:]
    o_ref[...] = jnp.concatenate([o_lo, o_hi], axis=1)


def _final(seg2, h2, s, b2):
    n = h2.shape[0]
    return pl.pallas_call(
        _final_kernel,
        grid=(n // _BLK,),
        in_specs=[
            pl.BlockSpec((2, _BLK, 64), lambda i: (0, i, 0)),
            pl.BlockSpec((_BLK, 128), lambda i: (i, 0)),
            pl.BlockSpec((_BLK, 1), lambda i: (i, 0)),
            pl.BlockSpec((1, 128), lambda i: (0, 0)),
        ],
        out_specs=pl.BlockSpec((_BLK, 128), lambda i: (i, 0)),
        out_shape=jax.ShapeDtypeStruct((n, 128), jnp.float32),
    )(seg2, h2, s, b2.reshape(1, 128))


def kernel(x, edge_index, W1, b1, W2, b2):
    src = edge_index[0]
    dst = edge_index[1]

    offs = (jnp.arange(_NC, dtype=jnp.int32) * _N)[:, None, None, None]
    src4 = src.reshape(1, _NS, _NBLK, _B) + offs
    dst3 = dst.reshape(_NS, _NBLK, _B)

    degp = _deg_pass(dst)
    h1 = _matmul(x, W1)
    s, h1sa, h1sb = _scale1(degp, h1)
    seg1a = _scatter_pass(h1sa, src4, dst3)
    seg1b = _scatter_pass(h1sb, src4, dst3)
    h2, h2s = _mid(seg1a, seg1b, h1, s, b1, W2)
    seg2 = _scatter_pass(h2s, src4, dst3)
    return _final(seg2, h2, s, b2)
